# baseline (speedup 1.0000x reference)
import functools

import jax
import jax.numpy as jnp
from jax import lax
from jax.experimental import pallas as pl
from jax.experimental.pallas import tpu as pltpu
from jax.experimental.pallas import tpu_sc as plsc

BATCH = 16384
EMBED_DIM = 64
NUM_ENT = 1000000
MAIN_ENT = 999424
TAIL_PAD = 640
NUM_REL = 1000
NUM_CORES = 2
NUM_SUBCORES = 16
ROWS_PER_TILE = BATCH // NUM_SUBCORES
DIMS_PER_CORE = EMBED_DIM // NUM_CORES
LANES = 16
IDX_CHUNK = 128
NUM_IDX_CHUNKS = ROWS_PER_TILE // IDX_CHUNK
GROUPS_PER_CHUNK = IDX_CHUNK // LANES


def _sc_partials_body(hid_hbm, rid_hbm, tid_hbm, entT_hbm, tailT_hbm,
                      relT_hbm, out_hbm, hid_v, rid_v, tid_v, relcol,
                      hbuf, tbuf, scores_v, spcol, sem_s, sem_g):
    cid = lax.axis_index("c")
    sid = lax.axis_index("s")

    pltpu.sync_copy(hid_hbm.at[sid], hid_v)
    pltpu.sync_copy(rid_hbm.at[sid], rid_v)
    pltpu.sync_copy(tid_hbm.at[sid], tid_v)

    def zinit(g, carry):
        scores_v[pl.ds(g * LANES, LANES)] = jnp.zeros((LANES,), jnp.float32)
        return carry
    lax.fori_loop(0, ROWS_PER_TILE // LANES, zinit, 0)

    def dim(d, carry):
        j = cid * DIMS_PER_CORE + d

        for t in range(8):
            off = t * 124928

            @pl.when(sid == t)
            def _stage(off=off):
                pltpu.async_copy(entT_hbm.at[j, pl.ds(off, 124928)],
                                 spcol.at[pl.ds(off, 124928)], sem_s).wait()

        @pl.when(sid == 8)
        def _stage_tail():
            pltpu.async_copy(tailT_hbm.at[j],
                             spcol.at[pl.ds(MAIN_ENT, TAIL_PAD)],
                             sem_s).wait()

        pltpu.sync_copy(relT_hbm.at[j], relcol)

        plsc.subcore_barrier()

        copies = []
        for k in range(NUM_IDX_CHUNKS):
            copies.append(pltpu.async_copy(spcol.at[hid_v.at[k]],
                                           hbuf.at[k], sem_g))
            copies.append(pltpu.async_copy(spcol.at[tid_v.at[k]],
                                           tbuf.at[k], sem_g))
        for c in copies:
            c.wait()

        for k in range(NUM_IDX_CHUNKS):
            for g in range(GROUPS_PER_CHUNK):
                sl = pl.ds(g * LANES, LANES)
                hv = hbuf[k, sl]
                tv = tbuf[k, sl]
                rv = plsc.load_gather(relcol, [rid_v[k, sl]])
                row0 = k * IDX_CHUNK + g * LANES
                scores_v[pl.ds(row0, LANES)] = (
                    scores_v[pl.ds(row0, LANES)] + hv * rv * tv)

        plsc.subcore_barrier()
        return carry

    lax.fori_loop(0, DIMS_PER_CORE, dim, 0)

    pltpu.sync_copy(scores_v, out_hbm.at[cid, sid])


_sc_partials = functools.partial(
    pl.kernel,
    mesh=plsc.VectorSubcoreMesh(core_axis_name="c", subcore_axis_name="s"),
    out_type=jax.ShapeDtypeStruct((NUM_CORES, NUM_SUBCORES, ROWS_PER_TILE),
                                  jnp.float32),
    scratch_types=[
        pltpu.VMEM((NUM_IDX_CHUNKS, IDX_CHUNK), jnp.int32),
        pltpu.VMEM((NUM_IDX_CHUNKS, IDX_CHUNK), jnp.int32),
        pltpu.VMEM((NUM_IDX_CHUNKS, IDX_CHUNK), jnp.int32),
        pltpu.VMEM((NUM_REL,), jnp.float32),
        pltpu.VMEM((NUM_IDX_CHUNKS, IDX_CHUNK), jnp.float32),
        pltpu.VMEM((NUM_IDX_CHUNKS, IDX_CHUNK), jnp.float32),
        pltpu.VMEM((ROWS_PER_TILE,), jnp.float32),
        pltpu.VMEM_SHARED((MAIN_ENT + TAIL_PAD,), jnp.float32),
        pltpu.SemaphoreType.DMA,
        pltpu.SemaphoreType.DMA,
    ],
    compiler_params=pltpu.CompilerParams(needs_layout_passes=False),
)(_sc_partials_body)


def _softmax_body(x_ref, o_ref):
    scores = x_ref[0] + x_ref[1]
    m = jnp.max(scores)
    e = jnp.exp(scores - m)
    o_ref[...] = e * (1.0 / jnp.sum(e))


_softmax = pl.pallas_call(
    _softmax_body,
    out_shape=jax.ShapeDtypeStruct((128, 128), jnp.float32),
)


def kernel(head_ids, rel_ids, tail_ids, entity_embeddings, relation_embeddings):
    hid = head_ids.astype(jnp.int32).reshape(NUM_SUBCORES, NUM_IDX_CHUNKS,
                                             IDX_CHUNK)
    rid = rel_ids.astype(jnp.int32).reshape(NUM_SUBCORES, NUM_IDX_CHUNKS,
                                            IDX_CHUNK)
    tid = tail_ids.astype(jnp.int32).reshape(NUM_SUBCORES, NUM_IDX_CHUNKS,
                                             IDX_CHUNK)
    entT = entity_embeddings.T
    relT = relation_embeddings.T
    tailT = jnp.pad(entT[:, MAIN_ENT:], ((0, 0), (0, TAIL_PAD - (NUM_ENT - MAIN_ENT))))
    partials = _sc_partials(hid, rid, tid, entT, tailT, relT)
    return _softmax(partials.reshape(2, 128, 128)).reshape(BATCH)
